# baseline (speedup 1.0000x reference)
import functools

import jax
import jax.numpy as jnp
from jax import lax
from jax.experimental import pallas as pl
from jax.experimental.pallas import tpu as pltpu
from jax.experimental.pallas import tpu_sc as plsc

_B = 16384
_D = 64
_NC = 2
_NS = 16
_NW = _NC * _NS
_BPW = _B // _NW
_AVG = 3.5
_L = 16
_ROWS_PER_STEP = 2048


def _tc_body(b_ref, x_ref, w_ref, o_ref):
    acc = jnp.dot(x_ref[...], w_ref[...], preferred_element_type=jnp.float32)
    o_ref[...] = acc + (b_ref[0] + _AVG)


@functools.partial(jax.jit, donate_argnums=())
def _tc_linear(x, w2, b):
    grid = (_B // _ROWS_PER_STEP,)
    return pl.pallas_call(
        _tc_body,
        grid=grid,
        in_specs=[
            pl.BlockSpec(memory_space=pltpu.SMEM),
            pl.BlockSpec((_ROWS_PER_STEP, _D), lambda i: (i, 0)),
            pl.BlockSpec((_D, 1), lambda i: (0, 0)),
        ],
        out_specs=pl.BlockSpec((_ROWS_PER_STEP, 1), lambda i: (i, 0)),
        out_shape=jax.ShapeDtypeStruct((_B, 1), jnp.float32),
    )(b, x, w2)


def _sc_body(lin_hbm, user_hbm, item_hbm, ubias_hbm, ibias_hbm,
             out_hbm, uidx_v, iidx_v, ub_v, ib_v, lin_v, out_v, sem):
    wid = lax.axis_index("s") * _NC + lax.axis_index("c")
    base = wid * _BPW

    pltpu.sync_copy(user_hbm.at[pl.ds(base, _BPW)], uidx_v)
    pltpu.sync_copy(item_hbm.at[pl.ds(base, _BPW)], iidx_v)
    ug = pltpu.async_copy(ubias_hbm.at[uidx_v], ub_v, sem)
    ig = pltpu.async_copy(ibias_hbm.at[iidx_v], ib_v, sem)
    pltpu.sync_copy(lin_hbm.at[pl.ds(base, _BPW)], lin_v)
    ug.wait()
    ig.wait()

    def group(g, _):
        c = g * _L
        out_v[pl.ds(c, _L)] = (lin_v[pl.ds(c, _L)]
                               + ub_v[pl.ds(c, _L)] + ib_v[pl.ds(c, _L)])
        return ()

    lax.fori_loop(0, _BPW // _L, group, ())

    pltpu.sync_copy(out_v, out_hbm.at[pl.ds(base, _BPW)])


@jax.jit
def _sc_combine(lin_flat, user, item, ubias_flat, ibias_flat):
    mesh = plsc.VectorSubcoreMesh(core_axis_name="c", subcore_axis_name="s")
    f = functools.partial(
        pl.kernel,
        out_type=jax.ShapeDtypeStruct((_B,), jnp.float32),
        mesh=mesh,
        scratch_types=[
            pltpu.VMEM((_BPW,), jnp.int32),
            pltpu.VMEM((_BPW,), jnp.int32),
            pltpu.VMEM((_BPW,), jnp.float32),
            pltpu.VMEM((_BPW,), jnp.float32),
            pltpu.VMEM((_BPW,), jnp.float32),
            pltpu.VMEM((_BPW,), jnp.float32),
            pltpu.SemaphoreType.DMA,
        ],
    )(_sc_body)
    return f(lin_flat, user, item, ubias_flat, ibias_flat)


def kernel(mlp_concat_emebd, user, item, W, b, user_bias, item_bias):
    lin = _tc_linear(mlp_concat_emebd, W.reshape(_D, 1), b)
    return _sc_combine(lin.reshape(-1), user.astype(jnp.int32),
                       item.astype(jnp.int32),
                       user_bias.reshape(-1), item_bias.reshape(-1))


# speedup vs baseline: 1.1494x; 1.1494x over previous
import functools

import jax
import jax.numpy as jnp
from jax import lax
from jax.experimental import pallas as pl
from jax.experimental.pallas import tpu as pltpu
from jax.experimental.pallas import tpu_sc as plsc

_B = 16384
_D = 64
_NC = 2
_NS = 16
_NW = _NC * _NS
_BPW = _B // _NW
_AVG = 3.5
_L = 16
_HALF = _BPW // 2


def _body(xt_hbm, user_hbm, item_hbm, w_hbm, bias16_hbm, ubias_hbm, ibias_hbm,
          out_hbm, uidx_v, iidx_v, ub_v, ib_v, x_v, w_v, b16_v, out_v,
          gsem, xsem0, xsem1):
    wid = lax.axis_index("s") * _NC + lax.axis_index("c")
    base = wid * _BPW

    x0 = pltpu.async_copy(xt_hbm.at[:, pl.ds(base, _HALF)],
                          x_v.at[:, pl.ds(0, _HALF)], xsem0)

    pltpu.sync_copy(user_hbm.at[pl.ds(base, _BPW)], uidx_v)
    pltpu.sync_copy(item_hbm.at[pl.ds(base, _BPW)], iidx_v)
    ug = pltpu.async_copy(ubias_hbm.at[uidx_v], ub_v, gsem)
    ig = pltpu.async_copy(ibias_hbm.at[iidx_v], ib_v, gsem)

    x1 = pltpu.async_copy(xt_hbm.at[:, pl.ds(base + _HALF, _HALF)],
                          x_v.at[:, pl.ds(_HALF, _HALF)], xsem1)

    pltpu.sync_copy(w_hbm, w_v)
    pltpu.sync_copy(bias16_hbm, b16_v)

    wv = [w_v[pl.ds(k * _L, _L)] for k in range(_D // _L)]
    bconst = b16_v[pl.ds(0, _L)]

    ug.wait()
    ig.wait()

    def group(g, _):
        c = g * _L
        acc = bconst + ub_v[pl.ds(c, _L)] + ib_v[pl.ds(c, _L)]
        for j in range(_D):
            acc = acc + x_v[j, pl.ds(c, _L)] * wv[j // _L][j % _L]
        out_v[pl.ds(c, _L)] = acc
        return ()

    x0.wait()
    lax.fori_loop(0, _HALF // _L, group, ())
    x1.wait()
    lax.fori_loop(_HALF // _L, _BPW // _L, group, ())

    pltpu.sync_copy(out_v, out_hbm.at[pl.ds(base, _BPW)])


@jax.jit
def _run(xt, user, item, w_flat, bias16, ubias_flat, ibias_flat):
    mesh = plsc.VectorSubcoreMesh(core_axis_name="c", subcore_axis_name="s")
    f = functools.partial(
        pl.kernel,
        out_type=jax.ShapeDtypeStruct((_B,), jnp.float32),
        mesh=mesh,
        scratch_types=[
            pltpu.VMEM((_BPW,), jnp.int32),
            pltpu.VMEM((_BPW,), jnp.int32),
            pltpu.VMEM((_BPW,), jnp.float32),
            pltpu.VMEM((_BPW,), jnp.float32),
            pltpu.VMEM((_D, _BPW), jnp.float32),
            pltpu.VMEM((_D,), jnp.float32),
            pltpu.VMEM((_L,), jnp.float32),
            pltpu.VMEM((_BPW,), jnp.float32),
            pltpu.SemaphoreType.DMA,
            pltpu.SemaphoreType.DMA,
            pltpu.SemaphoreType.DMA,
        ],
    )(_body)
    return f(xt, user, item, w_flat, bias16, ubias_flat, ibias_flat)


def kernel(mlp_concat_emebd, user, item, W, b, user_bias, item_bias):
    w_flat = W.reshape(-1)
    bias16 = jnp.broadcast_to(b.reshape(1) + _AVG, (_L,))
    return _run(mlp_concat_emebd.T, user.astype(jnp.int32),
                item.astype(jnp.int32), w_flat, bias16,
                user_bias.reshape(-1), item_bias.reshape(-1))


# speedup vs baseline: 1.1604x; 1.0095x over previous
import functools

import jax
import jax.numpy as jnp
from jax import lax
from jax.experimental import pallas as pl
from jax.experimental.pallas import tpu as pltpu
from jax.experimental.pallas import tpu_sc as plsc

_B = 16384
_D = 64
_NC = 2
_NS = 16
_NW = _NC * _NS
_BPW = _B // _NW
_AVG = 3.5
_L = 16
_HALF = _BPW // 2


def _body(xt_hbm, user_hbm, item_hbm, w_hbm, bias16_hbm, ubias_hbm, ibias_hbm,
          out_hbm, uidx_v, iidx_v, ub_v, ib_v, x_v, w_v, b16_v, out_v,
          gsem, xsem0, xsem1):
    wid = lax.axis_index("s") * _NC + lax.axis_index("c")
    base = wid * _BPW

    x0 = pltpu.async_copy(xt_hbm.at[:, pl.ds(base, _HALF)],
                          x_v.at[:, pl.ds(0, _HALF)], xsem0)

    pltpu.sync_copy(user_hbm.at[pl.ds(base, _BPW)], uidx_v)
    pltpu.sync_copy(item_hbm.at[pl.ds(base, _BPW)], iidx_v)
    ug = pltpu.async_copy(ubias_hbm.at[uidx_v], ub_v, gsem)
    ig = pltpu.async_copy(ibias_hbm.at[iidx_v], ib_v, gsem)

    x1 = pltpu.async_copy(xt_hbm.at[:, pl.ds(base + _HALF, _HALF)],
                          x_v.at[:, pl.ds(_HALF, _HALF)], xsem1)

    pltpu.sync_copy(w_hbm, w_v)
    pltpu.sync_copy(bias16_hbm, b16_v)

    wv = [w_v[pl.ds(k * _L, _L)] for k in range(_D // _L)]
    bconst = b16_v[pl.ds(0, _L)]

    ug.wait()
    ig.wait()

    nacc = 8

    def group(g, _):
        c = g * _L
        accs = [None] * nacc
        for j in range(_D):
            term = x_v[j, pl.ds(c, _L)] * wv[j // _L][j % _L]
            k = j % nacc
            accs[k] = term if accs[k] is None else accs[k] + term
        accs[0] = accs[0] + (bconst + ub_v[pl.ds(c, _L)] + ib_v[pl.ds(c, _L)])
        while len(accs) > 1:
            accs = [a + b for a, b in zip(accs[::2], accs[1::2])]
        out_v[pl.ds(c, _L)] = accs[0]
        return ()

    x0.wait()
    lax.fori_loop(0, _HALF // _L, group, ())
    x1.wait()
    lax.fori_loop(_HALF // _L, _BPW // _L, group, ())

    pltpu.sync_copy(out_v, out_hbm.at[pl.ds(base, _BPW)])


@jax.jit
def _run(xt, user, item, w_flat, bias16, ubias_flat, ibias_flat):
    mesh = plsc.VectorSubcoreMesh(core_axis_name="c", subcore_axis_name="s")
    f = functools.partial(
        pl.kernel,
        out_type=jax.ShapeDtypeStruct((_B,), jnp.float32),
        mesh=mesh,
        scratch_types=[
            pltpu.VMEM((_BPW,), jnp.int32),
            pltpu.VMEM((_BPW,), jnp.int32),
            pltpu.VMEM((_BPW,), jnp.float32),
            pltpu.VMEM((_BPW,), jnp.float32),
            pltpu.VMEM((_D, _BPW), jnp.float32),
            pltpu.VMEM((_D,), jnp.float32),
            pltpu.VMEM((_L,), jnp.float32),
            pltpu.VMEM((_BPW,), jnp.float32),
            pltpu.SemaphoreType.DMA,
            pltpu.SemaphoreType.DMA,
            pltpu.SemaphoreType.DMA,
        ],
    )(_body)
    return f(xt, user, item, w_flat, bias16, ubias_flat, ibias_flat)


def kernel(mlp_concat_emebd, user, item, W, b, user_bias, item_bias):
    w_flat = W.reshape(-1)
    bias16 = jnp.broadcast_to(b.reshape(1) + _AVG, (_L,))
    return _run(mlp_concat_emebd.T, user.astype(jnp.int32),
                item.astype(jnp.int32), w_flat, bias16,
                user_bias.reshape(-1), item_bias.reshape(-1))
